# baseline (speedup 1.0000x reference)
import functools

import jax
import jax.numpy as jnp
from jax import lax
from jax.experimental import pallas as pl
from jax.experimental.pallas import tpu as pltpu
from jax.experimental.pallas import tpu_sc as plsc

N = 10000
NP = 10240
HALF = NP // 2
E = 320000
D = 128
NC = 2
NS = 16
EPT = E // NS
CH = 128
NCH = -(-EPT // CH)
EPAD = NCH * CH
TRASH = HALF
AR = 5376
RPT = AR // NS
WPT = HALF // NS
RCH = 160

RB = 400
GRID = N // RB


def _sc_mesh():
    return plsc.VectorSubcoreMesh(core_axis_name="c", subcore_axis_name="s")


def _make_prop():
    @functools.partial(
        pl.kernel,
        mesh=_sc_mesh(),
        out_type=jax.ShapeDtypeStruct((NP, D), jnp.float32),
        scratch_types=[
            pltpu.VMEM((2, CH), jnp.int32),
            pltpu.VMEM((CH, D), jnp.float32),
            pltpu.VMEM((RCH, D), jnp.float32),
            pltpu.VMEM_SHARED((AR, D), jnp.float32),
            pltpu.SemaphoreType.DMA,
        ],
    )
    def prop(h_hbm, pidx_hbm, zeros_hbm, out_hbm,
             pbuf_v, rows_v, zbuf_v, acc_sh, sem):
        c = lax.axis_index("c")
        s = lax.axis_index("s")
        pltpu.sync_copy(zeros_hbm, zbuf_v)
        zb = s * RPT
        for r0, rn in ((0, RCH), (RCH, RCH), (2 * RCH, RPT - 2 * RCH)):
            pltpu.sync_copy(zbuf_v.at[pl.ds(0, rn)],
                            acc_sh.at[pl.ds(zb + r0, rn)])
        plsc.subcore_barrier()

        def body(j, carry):
            pltpu.sync_copy(pidx_hbm.at[c, s, j], pbuf_v)
            gather = pltpu.async_copy(h_hbm.at[pbuf_v.at[0]], rows_v, sem)
            gather.wait()
            pltpu.sync_copy(rows_v, acc_sh.at[pbuf_v.at[1]], add=True)
            return carry

        lax.fori_loop(0, NCH, body, 0)
        plsc.subcore_barrier()
        wb = s * WPT
        for i in range(2):
            r0 = wb + i * RCH
            pltpu.sync_copy(acc_sh.at[pl.ds(r0, RCH)], zbuf_v)
            pltpu.sync_copy(zbuf_v, out_hbm.at[pl.ds(c * HALF + r0, RCH)])

    return prop


def _dinv_block(degp):
    return lax.rsqrt(degp[:, 0:1] + 1.0)


def _pre_body(x_ref, w_ref, b_ref, degp_ref, out_ref):
    dinv = _dinv_block(degp_ref[...])
    h = jnp.dot(x_ref[...], w_ref[...], preferred_element_type=jnp.float32)
    out_ref[...] = (h + b_ref[...]) * dinv


def _mid_body(p_ref, prev_ref, degp_ref, out_ref, *, power):
    dinv = _dinv_block(degp_ref[...])
    scale = dinv * dinv if power == 2 else dinv
    out_ref[...] = (p_ref[...] + prev_ref[...]) * scale


def _layer_body(p_ref, prev_ref, degp_ref, w_ref, b_ref, out_ref):
    dinv = _dinv_block(degp_ref[...])
    h = (p_ref[...] + prev_ref[...]) * dinv
    r = jnp.maximum(h, 0.0)
    h2 = jnp.dot(r, w_ref[...], preferred_element_type=jnp.float32)
    out_ref[...] = (h2 + b_ref[...]) * dinv


_ROWS = pl.BlockSpec((RB, D), lambda i: (i, 0))
_MAT = pl.BlockSpec((D, D), lambda i: (0, 0))
_BIAS = pl.BlockSpec((1, D), lambda i: (0, 0))
_DEGP = pl.BlockSpec((RB, D), lambda i: (i, 0))
_OUT = jax.ShapeDtypeStruct((N, D), jnp.float32)


def _tc_pre(x, w1, b1, degp):
    return pl.pallas_call(
        _pre_body, grid=(GRID,),
        in_specs=[_ROWS, _MAT, _BIAS, _DEGP], out_specs=_ROWS,
        out_shape=_OUT,
    )(x, w1, b1, degp)


def _tc_mid(p, prev, degp, power):
    return pl.pallas_call(
        functools.partial(_mid_body, power=power), grid=(GRID,),
        in_specs=[_ROWS, _ROWS, _DEGP], out_specs=_ROWS,
        out_shape=_OUT,
    )(p, prev, degp)


def _tc_layer(p, prev, degp, w2, b2):
    return pl.pallas_call(
        _layer_body, grid=(GRID,),
        in_specs=[_ROWS, _ROWS, _DEGP, _MAT, _BIAS], out_specs=_ROWS,
        out_shape=_OUT,
    )(p, prev, degp, w2, b2)


def kernel(x, edge_index, layer_K, W1, b1, W2, b2):
    del layer_K
    pad = ((0, 0), (0, EPAD - EPT))
    srcf = jnp.pad(edge_index[0].reshape(NS, EPT), pad)
    dstf = jnp.pad(edge_index[1].reshape(NS, EPT), pad, constant_values=-1)
    lo = jnp.arange(NC, dtype=jnp.int32).reshape(NC, 1, 1) * HALF
    own = (dstf[None] >= lo) & (dstf[None] < lo + HALF)
    src = jnp.broadcast_to(srcf[None], (NC, NS, EPAD)).reshape(NC, NS, NCH, CH)
    tdst = jnp.where(own, dstf[None] - lo, TRASH).reshape(NC, NS, NCH, CH)
    pidx = jnp.stack([src, tdst], axis=3)
    pidx0 = jnp.stack([jnp.zeros_like(src), tdst], axis=3)
    zrows = jnp.zeros((RCH, D), jnp.float32)
    ones_h = jnp.ones((N, D), jnp.float32)
    b1r = b1.reshape(1, D)
    b2r = b2.reshape(1, D)

    prop_fn = _make_prop()

    degp = prop_fn(ones_h, pidx0, zrows)

    ha = _tc_pre(x, W1, b1r, degp)
    p = prop_fn(ha, pidx, zrows)
    ha = _tc_mid(p, ha, degp, power=2)
    p = prop_fn(ha, pidx, zrows)
    ha = _tc_layer(p, ha, degp, W2, b2r)
    p = prop_fn(ha, pidx, zrows)
    ha = _tc_mid(p, ha, degp, power=2)
    p = prop_fn(ha, pidx, zrows)
    return _tc_mid(p, ha, degp, power=1)


# speedup vs baseline: 9.9721x; 9.9721x over previous
import functools

import jax
import jax.numpy as jnp
from jax import lax
from jax.experimental import pallas as pl
from jax.experimental.pallas import tpu as pltpu
from jax.experimental.pallas import tpu_sc as plsc

N = 10000
NP = 10240
HALF = NP // 2
E = 320000
D = 128
NC = 2
NS = 16
EPT = E // NS
CH = 128
NCH = -(-EPT // CH)
EPAD = NCH * CH
TRASH = HALF
AR = 5376
RPT = AR // NS
WPT = HALF // NS
RCH = 160

RB = 400
GRID = N // RB


def _sc_mesh():
    return plsc.VectorSubcoreMesh(core_axis_name="c", subcore_axis_name="s")


def _make_prop():
    @functools.partial(
        pl.kernel,
        mesh=_sc_mesh(),
        out_type=jax.ShapeDtypeStruct((NP, D), jnp.float32),
        scratch_types=[
            pltpu.VMEM((2, CH), jnp.int32),
            pltpu.VMEM((CH, D), jnp.float32),
            pltpu.VMEM((RCH, D), jnp.float32),
            pltpu.VMEM_SHARED((AR, D), jnp.float32),
            pltpu.SemaphoreType.DMA,
        ],
    )
    def prop(h_hbm, pidx_hbm, zeros_hbm, out_hbm,
             pbuf_v, rows_v, zbuf_v, acc_sh, sem):
        c = lax.axis_index("c")
        s = lax.axis_index("s")
        pltpu.sync_copy(zeros_hbm, zbuf_v)
        zb = s * RPT
        for r0, rn in ((0, RCH), (RCH, RCH), (2 * RCH, RPT - 2 * RCH)):
            pltpu.sync_copy(zbuf_v.at[pl.ds(0, rn)],
                            acc_sh.at[pl.ds(zb + r0, rn)])
        plsc.subcore_barrier()

        def body(j, carry):
            pltpu.sync_copy(pidx_hbm.at[c, s, j], pbuf_v)
            gather = pltpu.async_copy(h_hbm.at[pbuf_v.at[0]], rows_v, sem)
            gather.wait()
            pltpu.sync_copy(rows_v, acc_sh.at[pbuf_v.at[1]], add=True)
            return carry

        lax.fori_loop(0, NCH, body, 0)
        plsc.subcore_barrier()
        wb = s * WPT
        for i in range(2):
            r0 = wb + i * RCH
            pltpu.sync_copy(acc_sh.at[pl.ds(r0, RCH)], zbuf_v)
            pltpu.sync_copy(zbuf_v, out_hbm.at[pl.ds(c * HALF + r0, RCH)])

    return prop


def _dinv_block(degp):
    return lax.rsqrt(degp[:, 0:1] + 1.0)


def _pre_body(x_ref, w_ref, b_ref, degp_ref, out_ref):
    dinv = _dinv_block(degp_ref[...])
    h = jnp.dot(x_ref[...], w_ref[...], preferred_element_type=jnp.float32)
    out_ref[...] = (h + b_ref[...]) * dinv


def _mid_body(p_ref, prev_ref, degp_ref, out_ref, *, power):
    dinv = _dinv_block(degp_ref[...])
    scale = dinv * dinv if power == 2 else dinv
    out_ref[...] = (p_ref[...] + prev_ref[...]) * scale


def _layer_body(p_ref, prev_ref, degp_ref, w_ref, b_ref, out_ref):
    dinv = _dinv_block(degp_ref[...])
    h = (p_ref[...] + prev_ref[...]) * dinv
    r = jnp.maximum(h, 0.0)
    h2 = jnp.dot(r, w_ref[...], preferred_element_type=jnp.float32)
    out_ref[...] = (h2 + b_ref[...]) * dinv


_ROWS = pl.BlockSpec((RB, D), lambda i: (i, 0))
_MAT = pl.BlockSpec((D, D), lambda i: (0, 0))
_BIAS = pl.BlockSpec((1, D), lambda i: (0, 0))
_DEGP = pl.BlockSpec((RB, D), lambda i: (i, 0))
_OUT = jax.ShapeDtypeStruct((N, D), jnp.float32)


def _tc_pre(x, w1, b1, degp):
    return pl.pallas_call(
        _pre_body, grid=(GRID,),
        in_specs=[_ROWS, _MAT, _BIAS, _DEGP], out_specs=_ROWS,
        out_shape=_OUT,
    )(x, w1, b1, degp)


def _tc_mid(p, prev, degp, power):
    return pl.pallas_call(
        functools.partial(_mid_body, power=power), grid=(GRID,),
        in_specs=[_ROWS, _ROWS, _DEGP], out_specs=_ROWS,
        out_shape=_OUT,
    )(p, prev, degp)


def _tc_layer(p, prev, degp, w2, b2):
    return pl.pallas_call(
        _layer_body, grid=(GRID,),
        in_specs=[_ROWS, _ROWS, _DEGP, _MAT, _BIAS], out_specs=_ROWS,
        out_shape=_OUT,
    )(p, prev, degp, w2, b2)


def kernel(x, edge_index, layer_K, W1, b1, W2, b2):
    del layer_K
    pad = ((0, 0), (0, EPAD - EPT))
    srcf = jnp.pad(edge_index[0].reshape(NS, EPT), pad)
    dstf = jnp.pad(edge_index[1].reshape(NS, EPT), pad, constant_values=-1)
    lo = jnp.arange(NC, dtype=jnp.int32).reshape(NC, 1, 1) * HALF
    own = (dstf[None] >= lo) & (dstf[None] < lo + HALF)
    src = jnp.broadcast_to(srcf[None], (NC, NS, EPAD)).reshape(NC, NS, NCH, CH)
    tdst = jnp.where(own, dstf[None] - lo, TRASH).reshape(NC, NS, NCH, CH)
    pidx = jnp.stack([src, tdst], axis=3)
    zrows = jnp.zeros((RCH, D), jnp.float32)
    ones_h = jnp.ones((N, D), jnp.float32)
    b1r = b1.reshape(1, D)
    b2r = b2.reshape(1, D)

    prop_fn = _make_prop()

    degp = prop_fn(ones_h, pidx, zrows)

    ha = _tc_pre(x, W1, b1r, degp)
    p = prop_fn(ha, pidx, zrows)
    ha = _tc_mid(p, ha, degp, power=2)
    p = prop_fn(ha, pidx, zrows)
    ha = _tc_layer(p, ha, degp, W2, b2r)
    p = prop_fn(ha, pidx, zrows)
    ha = _tc_mid(p, ha, degp, power=2)
    p = prop_fn(ha, pidx, zrows)
    return _tc_mid(p, ha, degp, power=1)
